# baseline (speedup 1.0000x reference)
import functools

import jax
import jax.numpy as jnp
from jax import lax
from jax.experimental import pallas as pl
from jax.experimental.pallas import tpu as pltpu
from jax.experimental.pallas import tpu_sc as plsc

_LANES = 16


def _dyn_gather(v, idx):
    dnums = lax.GatherDimensionNumbers(
        offset_dims=(), collapsed_slice_dims=(0,), start_index_map=(0,))
    return lax.gather(v, idx[:, None], dnums, (1,),
                      mode=lax.GatherScatterMode.PROMISE_IN_BOUNDS)


def _sc_fused_gather(X_domains, a_flat, table, arity):
    info = plsc.get_sparse_core_info()
    nc, ns = info.num_cores, info.num_subcores
    nw = nc * ns
    n = a_flat.shape[0]
    B = n // arity
    D = table.shape[1]
    npw = n // nw
    bpw = B // nw
    mesh = plsc.VectorSubcoreMesh(core_axis_name="c", subcore_axis_name="s",
                                  num_cores=nc)

    @functools.partial(
        pl.kernel,
        out_type=(jax.ShapeDtypeStruct((B, D), jnp.float32),
                  jax.ShapeDtypeStruct((B, D), jnp.float32)),
        mesh=mesh,
        scratch_types=[
            pltpu.VMEM((npw,), jnp.int32),
            pltpu.VMEM((npw,), jnp.int32),
            pltpu.VMEM((bpw,), jnp.int32),
            pltpu.VMEM((bpw,), jnp.int32),
            pltpu.VMEM((bpw, D), jnp.float32),
            pltpu.VMEM((bpw, D), jnp.float32),
            pltpu.SemaphoreType.DMA,
            pltpu.SemaphoreType.DMA,
        ],
        compiler_params=pltpu.CompilerParams(use_tc_tiling_on_sc=False),
    )
    def gather_kernel(xdom, a_hbm, tab, eh_out, et_out,
                      a_v, ci_v, ih_v, it_v, eh_v, et_v, sem_h, sem_t):
        wid = lax.axis_index("s") * nc + lax.axis_index("c")
        base = wid * bpw
        pltpu.sync_copy(a_hbm.at[pl.ds(wid * npw, npw)], a_v)
        pltpu.async_copy(xdom.at[a_v], ci_v, sem_h).wait()
        lane = lax.iota(jnp.int32, _LANES)
        perm_h = (lane % (_LANES // 2)) * arity
        perm_t = perm_h + 1
        lo = lane < (_LANES // 2)
        for i in range(bpw // _LANES):
            v0 = ci_v[pl.ds(i * 2 * _LANES, _LANES)]
            v1 = ci_v[pl.ds(i * 2 * _LANES + _LANES, _LANES)]
            h = jnp.where(lo, _dyn_gather(v0, perm_h), _dyn_gather(v1, perm_h))
            t = jnp.where(lo, _dyn_gather(v0, perm_t), _dyn_gather(v1, perm_t))
            ih_v[pl.ds(i * _LANES, _LANES)] = h
            it_v[pl.ds(i * _LANES, _LANES)] = t
        gh = pltpu.async_copy(tab.at[ih_v], eh_v, sem_h)
        gt = pltpu.async_copy(tab.at[it_v], et_v, sem_t)
        gh.wait()
        pltpu.sync_copy(eh_v, eh_out.at[pl.ds(base, bpw)])
        gt.wait()
        pltpu.sync_copy(et_v, et_out.at[pl.ds(base, bpw)])

    return gather_kernel(X_domains, a_flat, table)


def _mm_body(eh_ref, et_ref, w_ref, b_ref, o_ref):
    eh = eh_ref[...]
    et = et_ref[...]
    D = eh.shape[1]
    hp = jax.lax.Precision.HIGHEST
    acc = jnp.dot(eh, w_ref[0:D, :], precision=hp,
                  preferred_element_type=jnp.float32)
    acc = acc + jnp.dot(et, w_ref[D:2 * D, :], precision=hp,
                        preferred_element_type=jnp.float32)
    acc = acc + jnp.dot(eh * et, w_ref[2 * D:3 * D, :], precision=hp,
                        preferred_element_type=jnp.float32)
    o_ref[...] = jnp.tanh(acc + b_ref[...])


def _tc_embed(eh, et, W, b):
    B, D = eh.shape
    K, A = W.shape
    blk = 2048
    return pl.pallas_call(
        _mm_body,
        grid=(B // blk,),
        in_specs=[
            pl.BlockSpec((blk, D), lambda i: (i, 0)),
            pl.BlockSpec((blk, D), lambda i: (i, 0)),
            pl.BlockSpec((K, A), lambda i: (0, 0)),
            pl.BlockSpec((A,), lambda i: (0,)),
        ],
        out_specs=pl.BlockSpec((blk, A), lambda i: (i, 0)),
        out_shape=jax.ShapeDtypeStruct((B, A), jnp.float32),
    )(eh, et, W, b)


def kernel(X_domains, A_predicates, table, W, b):
    B, arity = A_predicates.shape
    a_flat = A_predicates.reshape(B * arity)
    eh, et = _sc_fused_gather(X_domains, a_flat, table, arity)
    return _tc_embed(eh, et, W, b)


# speedup vs baseline: 1.0040x; 1.0040x over previous
import functools

import jax
import jax.numpy as jnp
from jax import lax
from jax.experimental import pallas as pl
from jax.experimental.pallas import tpu as pltpu
from jax.experimental.pallas import tpu_sc as plsc


def _sc_fused_gather(X_domains, a_flat, table):
    info = plsc.get_sparse_core_info()
    nc, ns = info.num_cores, info.num_subcores
    nw = nc * ns
    n = a_flat.shape[0]
    D = table.shape[1]
    npw = n // nw
    mesh = plsc.VectorSubcoreMesh(core_axis_name="c", subcore_axis_name="s",
                                  num_cores=nc)

    @functools.partial(
        pl.kernel,
        out_type=jax.ShapeDtypeStruct((n, D), jnp.float32),
        mesh=mesh,
        scratch_types=[
            pltpu.VMEM((npw,), jnp.int32),
            pltpu.VMEM((npw,), jnp.int32),
            pltpu.VMEM((npw, D), jnp.float32),
            pltpu.SemaphoreType.DMA,
        ],
        compiler_params=pltpu.CompilerParams(use_tc_tiling_on_sc=False),
    )
    def gather_kernel(xdom, a_hbm, tab, rows_out, a_v, ci_v, rows_v, sem):
        wid = lax.axis_index("s") * nc + lax.axis_index("c")
        base = wid * npw
        pltpu.sync_copy(a_hbm.at[pl.ds(base, npw)], a_v)
        pltpu.async_copy(xdom.at[a_v], ci_v, sem).wait()
        pltpu.async_copy(tab.at[ci_v], rows_v, sem).wait()
        pltpu.sync_copy(rows_v, rows_out.at[pl.ds(base, npw)])

    return gather_kernel(X_domains, a_flat, table)


def _mm_body(eh_ref, et_ref, w_ref, b_ref, o_ref):
    eh = eh_ref[...]
    et = et_ref[...]
    D = eh.shape[1]
    hp = jax.lax.Precision.HIGHEST
    acc = jnp.dot(eh, w_ref[0:D, :], precision=hp,
                  preferred_element_type=jnp.float32)
    acc = acc + jnp.dot(et, w_ref[D:2 * D, :], precision=hp,
                        preferred_element_type=jnp.float32)
    acc = acc + jnp.dot(eh * et, w_ref[2 * D:3 * D, :], precision=hp,
                        preferred_element_type=jnp.float32)
    o_ref[...] = jnp.tanh(acc + b_ref[...])


def _tc_embed(rows, W, b):
    n, D = rows.shape
    B = n // 2
    K, A = W.shape
    blk = 2048
    nb = B // blk
    return pl.pallas_call(
        _mm_body,
        grid=(nb,),
        in_specs=[
            pl.BlockSpec((blk, D), lambda i: (i, 0)),
            pl.BlockSpec((blk, D), lambda i: (i + nb, 0)),
            pl.BlockSpec((K, A), lambda i: (0, 0)),
            pl.BlockSpec((A,), lambda i: (0,)),
        ],
        out_specs=pl.BlockSpec((blk, A), lambda i: (i, 0)),
        out_shape=jax.ShapeDtypeStruct((B, A), jnp.float32),
    )(rows, rows, W, b)


def kernel(X_domains, A_predicates, table, W, b):
    B, arity = A_predicates.shape
    a_flat = A_predicates.T.reshape(B * arity)
    rows = _sc_fused_gather(X_domains, a_flat, table)
    return _tc_embed(rows, W, b)
